# Initial kernel scaffold; baseline (speedup 1.0000x reference)
import jax
import jax.numpy as jnp
from jax.experimental import pallas as pl


def kernel(inputs):
    raise NotImplementedError("write your pallas kernel here")


# speedup vs baseline: 12.2762x; 12.2762x over previous
import jax
import jax.numpy as jnp
import numpy as np
from jax.experimental import pallas as pl
from jax.experimental.pallas import tpu as pltpu

_K = 102
_N = 1024
_MININT = np.int32(-2147483648)


def _topk_mean_body(x_ref, o_ref):
    x = x_ref[0]
    b = jax.lax.bitcast_convert_type(x, jnp.int32)
    key = jnp.where(b < 0, _MININT - b, b)

    def bit_step(i, prefix):
        bit = jnp.left_shift(jnp.int32(1), jnp.int32(31) - i)
        cand_u = prefix | bit
        cand_s = cand_u ^ _MININT
        cnt = jnp.sum((key >= cand_s).astype(jnp.int32), axis=0,
                      keepdims=True)
        return jnp.where(cnt >= _K, cand_u, prefix)

    prefix = jnp.zeros((1, x.shape[1]), jnp.int32)
    prefix = jax.lax.fori_loop(0, 32, bit_step, prefix, unroll=True)

    thr_s = prefix ^ _MININT
    gt = key > thr_s
    cnt_gt = jnp.sum(gt.astype(jnp.int32), axis=0)
    sum_gt = jnp.sum(jnp.where(gt, x, 0.0), axis=0)
    thr_b = jnp.where(thr_s < 0, _MININT - thr_s, thr_s)
    thr_f = jax.lax.bitcast_convert_type(thr_b, jnp.float32)[0]
    mean = (sum_gt + (_K - cnt_gt).astype(jnp.float32) * thr_f) / _K
    o_ref[0, 0] = mean


def kernel(inputs):
    B, H, W, C = inputs.shape
    x = inputs.reshape(B, H * W, C)
    out = pl.pallas_call(
        _topk_mean_body,
        grid=(B,),
        in_specs=[pl.BlockSpec((1, H * W, C), lambda i: (i, 0, 0))],
        out_specs=pl.BlockSpec((1, 1, C), lambda i: (i, 0, 0)),
        out_shape=jax.ShapeDtypeStruct((B, 1, C), jnp.float32),
    )(x)
    return out.reshape(B, C)
